# baseline (speedup 1.0000x reference)
import numpy as np
import jax
import jax.numpy as jnp
from jax.experimental import pallas as pl
from jax.experimental.pallas import tpu as pltpu

_K = 128
_N = 256
_HID = 512
_B = 1024
_BLK = 1024
_RATE = _K / _N
_THRESH = 0.5


def _build_info_mask(n, k):
    m = int(np.log2(n))
    z = np.array([0.5], dtype=np.float64)
    for _ in range(m):
        z = np.concatenate([2.0 * z - z * z, z * z])
    order = np.argsort(z, kind='stable')
    mask = np.zeros(n, dtype=bool)
    mask[order[:k]] = True
    return mask


_INFO_MASK = _build_info_mask(_N, _K)
_INFO_IDX = np.where(_INFO_MASK)[0]
_FROZEN = ~_INFO_MASK


def _gf2_generator(n):
    g = np.array([[1]], dtype=np.int64)
    f = np.array([[1, 1], [0, 1]], dtype=np.int64)
    for _ in range(int(np.log2(n))):
        g = np.kron(f, g) % 2
    return g


_G = _gf2_generator(_N)
_G_ENC = _G[:, _INFO_IDX].astype(np.float32)
_G_EXTRACT = _G[_INFO_IDX, :].astype(np.float32)


def _mod2(v):
    return v - 2.0 * jnp.floor(v * 0.5)


def _dot(a, b, ca, cb):
    return jax.lax.dot_general(
        a, b, dimension_numbers=(((ca,), (cb,)), ((), ())),
        preferred_element_type=jnp.float32)


def _xor(a, b):
    return a + b - 2.0 * a * b


def _xorb(a, b):
    return jax.lax.bitcast_convert_type(
        jax.lax.bitcast_convert_type(a, jnp.int32)
        ^ jax.lax.bitcast_convert_type(b, jnp.int32), jnp.float32)


def _decode(llr, frz):
    h = int(frz.shape[0])
    if not frz.any():
        return (llr < 0.0).astype(jnp.float32)
    if frz.all():
        return jnp.zeros_like(llr)
    if frz[:-1].all() and not frz[-1]:
        s = llr
        while s.shape[0] > 1:
            hh = s.shape[0] // 2
            s = s[:hh] + s[hh:]
        bit = (s < 0.0).astype(jnp.float32)
        return jnp.broadcast_to(bit, llr.shape)
    if frz[0] and not frz[1:].any():
        hd = (llr < 0.0).astype(jnp.float32)
        mag = jnp.abs(llr)
        mn = jnp.min(mag, axis=0, keepdims=True)
        iota = jax.lax.broadcasted_iota(jnp.int32, llr.shape, 0)
        idx = jnp.min(jnp.where(mag == mn, iota, jnp.int32(h)),
                      axis=0, keepdims=True)
        first = (iota == idx).astype(jnp.float32)
        par = jnp.sum(hd, axis=0, keepdims=True)
        par = par - 2.0 * jnp.floor(par * 0.5)
        return _xorb(hd, first * par)
    half = h // 2
    lo = llr[:half]
    hi = llr[half:]
    ai = jax.lax.bitcast_convert_type(lo, jnp.int32)
    bi = jax.lax.bitcast_convert_type(hi, jnp.int32)
    sgn = (ai ^ bi) & jnp.int32(-2147483648)
    mag = jnp.minimum(ai & jnp.int32(2147483647), bi & jnp.int32(2147483647))
    f = jax.lax.bitcast_convert_type(sgn | mag, jnp.float32)
    x1 = _decode(f, frz[:half])
    flip = jax.lax.shift_left(
        jax.lax.bitcast_convert_type(x1, jnp.int32), jnp.int32(8))
    g = hi + jax.lax.bitcast_convert_type(ai ^ flip, jnp.float32)
    x2 = _decode(g, frz[half:])
    return jnp.concatenate([_xorb(x1, x2), x2], axis=0)


def _body(u_ref, a_ref, nr_ref, ni_ref, scal_ref, W1_ref, b1_ref,
          W2x_ref, b2x_ref, W2p_ref, b2p_ref, genc_ref, gext_ref,
          ct_ref, uh_ref, ch_ref, p_ref, ah_ref,
          W1v, W2xv, gextv, ctv, s_w1, s_w2, s_gx, s_ct):
    no = scal_ref[0, 0]
    sn = scal_ref[0, 1]
    bf16 = jnp.bfloat16
    genc = genc_ref[...]

    c_w1 = pltpu.make_async_copy(W1_ref, W1v, s_w1)
    c_w1.start()
    c_w2 = pltpu.make_async_copy(W2x_ref.at[:, pl.ds(0, _N)], W2xv, s_w2)
    c_w2.start()
    c_gx = pltpu.make_async_copy(gext_ref, gextv, s_gx)
    c_gx.start()

    c_true = _mod2(_dot(u_ref[...].astype(bf16), genc, 1, 1))
    ctv[...] = c_true
    c_ct = pltpu.make_async_copy(ctv, ct_ref, s_ct)
    c_ct.start()

    x = (1.0 - 2.0 * c_true) * a_ref[...]
    y_r = x + nr_ref[...] * sn
    y_i = ni_ref[...] * sn
    y = jnp.concatenate([y_r, y_i], axis=1)

    c_w1.wait()
    hh = jnp.maximum(_dot(y, W1v[...], 1, 0) + b1_ref[...], 0.0)
    p = jax.nn.sigmoid(_dot(hh, W2p_ref[...], 1, 0) + b2p_ref[0, 0])
    a_hat = (p > _THRESH).astype(jnp.float32)
    p_ref[...] = p
    ah_ref[...] = a_hat

    hh3 = hh.reshape(8, _BLK // 8, _HID)
    c_w2.wait()
    y_hat_r = jax.lax.dot_general(
        W2xv[...], hh3,
        dimension_numbers=(((0,), (2,)), ((), ())),
        preferred_element_type=jnp.float32)
    y_hat_r = y_hat_r + b2x_ref[...].reshape(_N, 1, 1)

    llr = 4.0 * y_hat_r / no
    x_all = _decode(llr, _FROZEN)
    c_gx.wait()
    u_hat3 = _mod2(jax.lax.dot_general(
        x_all.astype(bf16), gextv[...],
        dimension_numbers=(((0,), (1,)), ((), ())),
        preferred_element_type=jnp.float32))
    u_hat = u_hat3.reshape(_BLK, _K) * a_hat
    uh_ref[...] = u_hat
    ch_ref[...] = _mod2(_dot(u_hat.astype(bf16), genc, 1, 1))
    c_ct.wait()

def kernel(u, a_true, noise_r, noise_i, ebno_db, W1, b1, W2x, b2x, W2p, b2p):
    no = 1.0 / (jnp.power(10.0, ebno_db / 10.0) * 1.0 * _RATE)
    sn = jnp.sqrt(no / 2.0)
    scal = jnp.stack([no, sn]).reshape(1, 2).astype(jnp.float32)

    f32 = jnp.float32
    out_shape = (
        jax.ShapeDtypeStruct((_B, _N), f32),
        jax.ShapeDtypeStruct((_B, _K), f32),
        jax.ShapeDtypeStruct((_B, _N), f32),
        jax.ShapeDtypeStruct((_B, 1), f32),
        jax.ShapeDtypeStruct((_B, 1), f32),
    )

    def _batch(shape):
        return pl.BlockSpec((_BLK,) + shape[1:], lambda i: (i,) + (0,) * (len(shape) - 1))

    def _whole(shape):
        return pl.BlockSpec(shape, lambda i: (0,) * len(shape))

    in_specs = [
        _batch((_B, _K)),
        _batch((_B, 1)),
        _batch((_B, _N)),
        _batch((_B, _N)),
        _whole((1, 2)),
        pl.BlockSpec(memory_space=pl.ANY),
        _whole((1, _HID)),
        pl.BlockSpec(memory_space=pl.ANY),
        pl.BlockSpec((_N, 1), lambda i: (0, 0)),
        _whole((_HID, 1)),
        _whole((1, 1)),
        _whole((_N, _K)),
        pl.BlockSpec(memory_space=pl.ANY),
    ]
    out_specs = (
        pl.BlockSpec(memory_space=pl.ANY),
        _batch((_B, _K)),
        _batch((_B, _N)),
        _batch((_B, 1)),
        _batch((_B, 1)),
    )
    ct, uh, ch, p, ah = pl.pallas_call(
        _body,
        grid=(_B // _BLK,),
        in_specs=in_specs,
        out_specs=out_specs,
        out_shape=out_shape,
        scratch_shapes=[
            pltpu.VMEM((2 * _N, _HID), jnp.float32),
            pltpu.VMEM((_HID, _N), jnp.float32),
            pltpu.VMEM((_K, _N), jnp.bfloat16),
            pltpu.VMEM((_B, _N), jnp.float32),
            pltpu.SemaphoreType.DMA,
            pltpu.SemaphoreType.DMA,
            pltpu.SemaphoreType.DMA,
            pltpu.SemaphoreType.DMA,
        ],
        compiler_params=pltpu.CompilerParams(
            dimension_semantics=("arbitrary",)),
    )(
        u, a_true, noise_r, noise_i, scal,
        W1, b1.reshape(1, _HID), W2x, b2x.reshape(2 * _N, 1),
        W2p, b2p.reshape(1, 1),
        jnp.asarray(_G_ENC, dtype=jnp.bfloat16),
        jnp.asarray(_G_EXTRACT, dtype=jnp.bfloat16))

    return (u, uh, ct, ch, a_true, p, ah)


# speedup vs baseline: 1.0536x; 1.0536x over previous
import numpy as np
import jax
import jax.numpy as jnp
from jax.experimental import pallas as pl
from jax.experimental.pallas import tpu as pltpu

_K = 128
_N = 256
_HID = 512
_B = 1024
_BLK = 1024
_RATE = _K / _N
_THRESH = 0.5


def _build_info_mask(n, k):
    m = int(np.log2(n))
    z = np.array([0.5], dtype=np.float64)
    for _ in range(m):
        z = np.concatenate([2.0 * z - z * z, z * z])
    order = np.argsort(z, kind='stable')
    mask = np.zeros(n, dtype=bool)
    mask[order[:k]] = True
    return mask


_INFO_MASK = _build_info_mask(_N, _K)
_INFO_IDX = np.where(_INFO_MASK)[0]
_FROZEN = ~_INFO_MASK


def _gf2_generator(n):
    g = np.array([[1]], dtype=np.int64)
    f = np.array([[1, 1], [0, 1]], dtype=np.int64)
    for _ in range(int(np.log2(n))):
        g = np.kron(f, g) % 2
    return g


_G = _gf2_generator(_N)
_G_ENC = _G[:, _INFO_IDX].astype(np.float32)
_G_EXTRACT = _G[_INFO_IDX, :].astype(np.float32)


def _mod2(v):
    return v - 2.0 * jnp.floor(v * 0.5)


def _dot(a, b, ca, cb):
    return jax.lax.dot_general(
        a, b, dimension_numbers=(((ca,), (cb,)), ((), ())),
        preferred_element_type=jnp.float32)


def _xor(a, b):
    return a + b - 2.0 * a * b


def _xorb(a, b):
    return jax.lax.bitcast_convert_type(
        jax.lax.bitcast_convert_type(a, jnp.int32)
        ^ jax.lax.bitcast_convert_type(b, jnp.int32), jnp.float32)


def _decode(llr, frz):
    h = int(frz.shape[0])
    if not frz.any():
        return (llr < 0.0).astype(jnp.float32)
    if frz.all():
        return jnp.zeros_like(llr)
    if frz[:-1].all() and not frz[-1]:
        s = llr
        while s.shape[0] > 1:
            hh = s.shape[0] // 2
            s = s[:hh] + s[hh:]
        bit = (s < 0.0).astype(jnp.float32)
        return jnp.broadcast_to(bit, llr.shape)
    if frz[0] and not frz[1:].any():
        hd = (llr < 0.0).astype(jnp.float32)
        mag = jnp.abs(llr)
        mn = jnp.min(mag, axis=0, keepdims=True)
        iota = jax.lax.broadcasted_iota(jnp.int32, llr.shape, 0)
        idx = jnp.min(jnp.where(mag == mn, iota, jnp.int32(h)),
                      axis=0, keepdims=True)
        first = (iota == idx).astype(jnp.float32)
        par = jnp.sum(hd, axis=0, keepdims=True)
        par = par - 2.0 * jnp.floor(par * 0.5)
        return _xorb(hd, first * par)
    half = h // 2
    lo = llr[:half]
    hi = llr[half:]
    ai = jax.lax.bitcast_convert_type(lo, jnp.int32)
    bi = jax.lax.bitcast_convert_type(hi, jnp.int32)
    sgn = (ai ^ bi) & jnp.int32(-2147483648)
    mag = jnp.minimum(ai & jnp.int32(2147483647), bi & jnp.int32(2147483647))
    f = jax.lax.bitcast_convert_type(sgn | mag, jnp.float32)
    x1 = _decode(f, frz[:half])
    flip = jax.lax.shift_left(
        jax.lax.bitcast_convert_type(x1, jnp.int32), jnp.int32(8))
    g = hi + jax.lax.bitcast_convert_type(ai ^ flip, jnp.float32)
    x2 = _decode(g, frz[half:])
    return jnp.concatenate([_xorb(x1, x2), x2], axis=0)


def _body(u_ref, a_ref, nr_ref, ni_ref, scal_ref, W1_ref, b1_ref,
          W2x_ref, b2x_ref, W2p_ref, b2p_ref, genc_ref, gext_ref,
          ct_ref, uh_ref, ch_ref, p_ref, ah_ref, W1v, s_w1):
    no = scal_ref[0, 0]
    sn = scal_ref[0, 1]
    bf16 = jnp.bfloat16
    genc = genc_ref[...]

    c_w1 = pltpu.make_async_copy(W1_ref, W1v, s_w1)
    c_w1.start()

    c_true = _mod2(_dot(u_ref[...].astype(bf16), genc, 1, 1))
    ct_ref[...] = c_true

    x = (1.0 - 2.0 * c_true) * a_ref[...]
    y_r = x + nr_ref[...] * sn
    y_i = ni_ref[...] * sn
    y = jnp.concatenate([y_r, y_i], axis=1)

    c_w1.wait()
    hh = jnp.maximum(_dot(y, W1v[...], 1, 0) + b1_ref[...], 0.0)
    p = jax.nn.sigmoid(_dot(hh, W2p_ref[...], 1, 0) + b2p_ref[0, 0])
    a_hat = (p > _THRESH).astype(jnp.float32)
    p_ref[...] = p
    ah_ref[...] = a_hat

    hh3 = hh.reshape(8, _BLK // 8, _HID)
    y_hat_r = jax.lax.dot_general(
        W2x_ref[...], hh3,
        dimension_numbers=(((0,), (2,)), ((), ())),
        preferred_element_type=jnp.float32)
    y_hat_r = y_hat_r + b2x_ref[...].reshape(_N, 1, 1)

    llr = 4.0 * y_hat_r / no
    x_all = _decode(llr, _FROZEN)
    u_hat3 = _mod2(jax.lax.dot_general(
        x_all.astype(bf16), gext_ref[...],
        dimension_numbers=(((0,), (1,)), ((), ())),
        preferred_element_type=jnp.float32))
    u_hat = u_hat3.reshape(_BLK, _K) * a_hat
    uh_ref[...] = u_hat
    ch_ref[...] = _mod2(_dot(u_hat.astype(bf16), genc, 1, 1))

def kernel(u, a_true, noise_r, noise_i, ebno_db, W1, b1, W2x, b2x, W2p, b2p):
    no = 1.0 / (jnp.power(10.0, ebno_db / 10.0) * 1.0 * _RATE)
    sn = jnp.sqrt(no / 2.0)
    scal = jnp.stack([no, sn]).reshape(1, 2).astype(jnp.float32)

    f32 = jnp.float32
    out_shape = (
        jax.ShapeDtypeStruct((_B, _N), f32),
        jax.ShapeDtypeStruct((_B, _K), f32),
        jax.ShapeDtypeStruct((_B, _N), f32),
        jax.ShapeDtypeStruct((_B, 1), f32),
        jax.ShapeDtypeStruct((_B, 1), f32),
    )

    def _batch(shape):
        return pl.BlockSpec((_BLK,) + shape[1:], lambda i: (i,) + (0,) * (len(shape) - 1))

    def _whole(shape):
        return pl.BlockSpec(shape, lambda i: (0,) * len(shape))

    in_specs = [
        _batch((_B, _K)),
        _batch((_B, 1)),
        _batch((_B, _N)),
        _batch((_B, _N)),
        _whole((1, 2)),
        pl.BlockSpec(memory_space=pl.ANY),
        _whole((1, _HID)),
        pl.BlockSpec((_HID, _N), lambda i: (0, 0)),
        pl.BlockSpec((_N, 1), lambda i: (0, 0)),
        _whole((_HID, 1)),
        _whole((1, 1)),
        _whole((_N, _K)),
        _whole((_K, _N)),
    ]
    out_specs = (
        _batch((_B, _N)),
        _batch((_B, _K)),
        _batch((_B, _N)),
        _batch((_B, 1)),
        _batch((_B, 1)),
    )
    ct, uh, ch, p, ah = pl.pallas_call(
        _body,
        grid=(_B // _BLK,),
        in_specs=in_specs,
        out_specs=out_specs,
        out_shape=out_shape,
        scratch_shapes=[
            pltpu.VMEM((2 * _N, _HID), jnp.float32),
            pltpu.SemaphoreType.DMA,
        ],
        compiler_params=pltpu.CompilerParams(
            dimension_semantics=("arbitrary",)),
    )(
        u, a_true, noise_r, noise_i, scal,
        W1, b1.reshape(1, _HID), W2x, b2x.reshape(2 * _N, 1),
        W2p, b2p.reshape(1, 1),
        jnp.asarray(_G_ENC, dtype=jnp.bfloat16),
        jnp.asarray(_G_EXTRACT, dtype=jnp.bfloat16))

    return (u, uh, ct, ch, a_true, p, ah)


# speedup vs baseline: 1.0939x; 1.0382x over previous
import numpy as np
import jax
import jax.numpy as jnp
from jax.experimental import pallas as pl
from jax.experimental.pallas import tpu as pltpu

_K = 128
_N = 256
_HID = 512
_B = 1024
_BLK = 1024
_RATE = _K / _N
_THRESH = 0.5


def _build_info_mask(n, k):
    m = int(np.log2(n))
    z = np.array([0.5], dtype=np.float64)
    for _ in range(m):
        z = np.concatenate([2.0 * z - z * z, z * z])
    order = np.argsort(z, kind='stable')
    mask = np.zeros(n, dtype=bool)
    mask[order[:k]] = True
    return mask


_INFO_MASK = _build_info_mask(_N, _K)
_INFO_IDX = np.where(_INFO_MASK)[0]
_FROZEN = ~_INFO_MASK


def _gf2_generator(n):
    g = np.array([[1]], dtype=np.int64)
    f = np.array([[1, 1], [0, 1]], dtype=np.int64)
    for _ in range(int(np.log2(n))):
        g = np.kron(f, g) % 2
    return g


_G = _gf2_generator(_N)
_G_ENC = _G[:, _INFO_IDX].astype(np.float32)
_G_EXTRACT = _G[_INFO_IDX, :].astype(np.float32)


def _mod2(v):
    return v - 2.0 * jnp.floor(v * 0.5)


def _dot(a, b, ca, cb):
    return jax.lax.dot_general(
        a, b, dimension_numbers=(((ca,), (cb,)), ((), ())),
        preferred_element_type=jnp.float32)


def _xor(a, b):
    return a + b - 2.0 * a * b


def _xorb(a, b):
    return jax.lax.bitcast_convert_type(
        jax.lax.bitcast_convert_type(a, jnp.int32)
        ^ jax.lax.bitcast_convert_type(b, jnp.int32), jnp.float32)


def _decode(llr, frz):
    h = int(frz.shape[0])
    if not frz.any():
        return (llr < 0.0).astype(jnp.float32)
    if frz.all():
        return jnp.zeros_like(llr)
    if frz[:-1].all() and not frz[-1]:
        s = llr
        while s.shape[0] > 1:
            hh = s.shape[0] // 2
            s = s[:hh] + s[hh:]
        bit = (s < 0.0).astype(jnp.float32)
        return jnp.broadcast_to(bit, llr.shape)
    if frz[0] and not frz[1:].any():
        hd = (llr < 0.0).astype(jnp.float32)
        mag = jnp.abs(llr)
        mn = jnp.min(mag, axis=0, keepdims=True)
        iota = jax.lax.broadcasted_iota(jnp.int32, llr.shape, 0)
        idx = jnp.min(jnp.where(mag == mn, iota, jnp.int32(h)),
                      axis=0, keepdims=True)
        first = (iota == idx).astype(jnp.float32)
        par = jnp.sum(hd, axis=0, keepdims=True)
        par = par - 2.0 * jnp.floor(par * 0.5)
        return _xorb(hd, first * par)
    half = h // 2
    lo = llr[:half]
    hi = llr[half:]
    ai = jax.lax.bitcast_convert_type(lo, jnp.int32)
    bi = jax.lax.bitcast_convert_type(hi, jnp.int32)
    sgn = (ai ^ bi) & jnp.int32(-2147483648)
    mag = jnp.minimum(ai & jnp.int32(2147483647), bi & jnp.int32(2147483647))
    f = jax.lax.bitcast_convert_type(sgn | mag, jnp.float32)
    x1 = _decode(f, frz[:half])
    flip = jax.lax.shift_left(
        jax.lax.bitcast_convert_type(x1, jnp.int32), jnp.int32(8))
    g = hi + jax.lax.bitcast_convert_type(ai ^ flip, jnp.float32)
    x2 = _decode(g, frz[half:])
    return jnp.concatenate([_xorb(x1, x2), x2], axis=0)


def _body(u_ref, a_ref, nr_ref, ni_ref, scal_ref, W1_ref, b1_ref,
          W2x_ref, b2x_ref, W2p_ref, b2p_ref, genc_ref, gext_ref,
          ct_ref, uh_ref, ch_ref, p_ref, ah_ref):
    no = scal_ref[0, 0]
    sn = scal_ref[0, 1]
    bf16 = jnp.bfloat16
    genc = genc_ref[...]

    c_true = _mod2(_dot(u_ref[...].astype(bf16), genc, 1, 1))
    ct_ref[...] = c_true

    x = (1.0 - 2.0 * c_true) * a_ref[...]
    y_r = x + nr_ref[...] * sn
    y_i = ni_ref[...] * sn
    y = jnp.concatenate([y_r, y_i], axis=1)

    hh = jnp.maximum(_dot(y, W1_ref[...], 1, 0) + b1_ref[...], 0.0)
    p = jax.nn.sigmoid(_dot(hh, W2p_ref[...], 1, 0) + b2p_ref[0, 0])
    a_hat = (p > _THRESH).astype(jnp.float32)
    p_ref[...] = p
    ah_ref[...] = a_hat

    hh3 = hh.reshape(8, _BLK // 8, _HID)
    y_hat_r = jax.lax.dot_general(
        W2x_ref[...], hh3,
        dimension_numbers=(((0,), (2,)), ((), ())),
        preferred_element_type=jnp.float32)
    y_hat_r = y_hat_r + b2x_ref[...].reshape(_N, 1, 1)

    llr = 4.0 * y_hat_r / no
    x_all = _decode(llr, _FROZEN)
    u_hat3 = _mod2(jax.lax.dot_general(
        x_all.astype(bf16), gext_ref[...],
        dimension_numbers=(((0,), (1,)), ((), ())),
        preferred_element_type=jnp.float32))
    u_hat = u_hat3.reshape(_BLK, _K) * a_hat
    uh_ref[...] = u_hat
    ch_ref[...] = _mod2(_dot(u_hat.astype(bf16), genc, 1, 1))

def kernel(u, a_true, noise_r, noise_i, ebno_db, W1, b1, W2x, b2x, W2p, b2p):
    no = 1.0 / (jnp.power(10.0, ebno_db / 10.0) * 1.0 * _RATE)
    sn = jnp.sqrt(no / 2.0)
    scal = jnp.stack([no, sn]).reshape(1, 2).astype(jnp.float32)

    f32 = jnp.float32
    out_shape = (
        jax.ShapeDtypeStruct((_B, _N), f32),
        jax.ShapeDtypeStruct((_B, _K), f32),
        jax.ShapeDtypeStruct((_B, _N), f32),
        jax.ShapeDtypeStruct((_B, 1), f32),
        jax.ShapeDtypeStruct((_B, 1), f32),
    )

    def _batch(shape):
        return pl.BlockSpec((_BLK,) + shape[1:], lambda i: (i,) + (0,) * (len(shape) - 1))

    def _whole(shape):
        return pl.BlockSpec(shape, lambda i: (0,) * len(shape))

    in_specs = [
        _batch((_B, _K)),
        _batch((_B, 1)),
        _batch((_B, _N)),
        _batch((_B, _N)),
        _whole((1, 2)),
        _whole((2 * _N, _HID)),
        _whole((1, _HID)),
        pl.BlockSpec((_HID, _N), lambda i: (0, 0)),
        pl.BlockSpec((_N, 1), lambda i: (0, 0)),
        _whole((_HID, 1)),
        _whole((1, 1)),
        _whole((_N, _K)),
        _whole((_K, _N)),
    ]
    out_specs = (
        _batch((_B, _N)),
        _batch((_B, _K)),
        _batch((_B, _N)),
        _batch((_B, 1)),
        _batch((_B, 1)),
    )
    ct, uh, ch, p, ah = pl.pallas_call(
        _body,
        grid=(_B // _BLK,),
        in_specs=in_specs,
        out_specs=out_specs,
        out_shape=out_shape,
        compiler_params=pltpu.CompilerParams(
            dimension_semantics=("arbitrary",)),
    )(
        u, a_true, noise_r, noise_i, scal,
        W1, b1.reshape(1, _HID), W2x, b2x.reshape(2 * _N, 1),
        W2p, b2p.reshape(1, 1),
        jnp.asarray(_G_ENC, dtype=jnp.bfloat16),
        jnp.asarray(_G_EXTRACT, dtype=jnp.bfloat16))

    return (u, uh, ct, ch, a_true, p, ah)
